# Initial kernel scaffold; baseline (speedup 1.0000x reference)
import jax
import jax.numpy as jnp
from jax.experimental import pallas as pl


def kernel(z, edge_index, edge_type, type_emb, W1, b1, W2, b2, Wm1, bm1, Wm2, bm2, gamma, beta):
    raise NotImplementedError("write your pallas kernel here")


# speedup vs baseline: 6.0751x; 6.0751x over previous
import functools

import jax
import jax.numpy as jnp
from jax import lax
from jax.experimental import pallas as pl
from jax.experimental.pallas import tpu as pltpu
from jax.experimental.pallas import tpu_sc as plsc

N = 10000
E = 320000
D = 128
TE = 8
ET_TILES = 125
T = E // ET_TILES

NC = 2
NS = 16
NW = NC * NS
EPW = E // NW
CH = 80
NCHUNK = EPW // CH
NPAD = 10240
SL = NPAD // NS

_mesh = plsc.VectorSubcoreMesh(core_axis_name="c", subcore_axis_name="s",
                               num_cores=NC, num_subcores=NS)
_sc_params = pltpu.CompilerParams(needs_layout_passes=False)


def _wid():
    return lax.axis_index("s") * NC + lax.axis_index("c")


def _node_mlp_body(z_ref, wm1_ref, bm1_ref, wm2_ref, bm2_ref, zm_ref):
    z = z_ref[...]
    h = jnp.maximum(jnp.dot(z, wm1_ref[...], preferred_element_type=jnp.float32)
                    + bm1_ref[...], 0.0)
    m = jnp.dot(h, wm2_ref[...], preferred_element_type=jnp.float32) + bm2_ref[...]
    zm_ref[:, :D] = z
    zm_ref[:, D:] = m


def _node_mlp(z, Wm1, bm1, Wm2, bm2):
    return pl.pallas_call(
        _node_mlp_body,
        out_shape=jax.ShapeDtypeStruct((N, 2 * D), jnp.float32),
    )(z, Wm1, bm1.reshape(1, D), Wm2, bm2.reshape(1, D))


def _gather_body(z_hbm, zm_hbm, src_hbm, dst_hbm, zs_out, zd_out,
                 sidx_v, didx_v, bufa, bufb, sema, semb):
    wid = _wid()
    pltpu.sync_copy(src_hbm.at[wid], sidx_v)
    pltpu.sync_copy(dst_hbm.at[wid], didx_v)
    base = wid * EPW

    def body(j, carry):
        a = pltpu.async_copy(z_hbm.at[sidx_v.at[j]], bufa, sema)
        b = pltpu.async_copy(zm_hbm.at[didx_v.at[j]], bufb, semb)
        a.wait()
        b.wait()
        row0 = base + j * CH
        pltpu.sync_copy(bufa, zs_out.at[pl.ds(row0, CH)])
        pltpu.sync_copy(bufb, zd_out.at[pl.ds(row0, CH)])
        return carry

    lax.fori_loop(0, NCHUNK, body, 0)


_sc_gather = functools.partial(
    pl.kernel,
    _gather_body,
    out_type=(jax.ShapeDtypeStruct((E, D), jnp.float32),
              jax.ShapeDtypeStruct((E, 2 * D), jnp.float32)),
    mesh=_mesh,
    compiler_params=_sc_params,
    scratch_types=[
        pltpu.VMEM((NCHUNK, CH), jnp.int32),
        pltpu.VMEM((NCHUNK, CH), jnp.int32),
        pltpu.VMEM((CH, D), jnp.float32),
        pltpu.VMEM((CH, 2 * D), jnp.float32),
        pltpu.SemaphoreType.DMA,
        pltpu.SemaphoreType.DMA,
    ],
)()


def _edge_mlp_body(zs_ref, zd_ref, etf_ref, w1_ref, b1_ref, w2_ref, b2_ref,
                   temb_ref, ew_ref):
    zs = zs_ref[...]
    d = jnp.abs(zs - zd_ref[...])
    w1a = w1_ref[0:D, :]
    w1b = w1_ref[D:2 * D, :]
    w1c = w1_ref[2 * D:3 * D, :]
    w1d = w1_ref[3 * D:3 * D + TE, :]
    t2 = jnp.dot(temb_ref[...], w1d, preferred_element_type=jnp.float32)
    pre = (jnp.dot(zs, w1a, preferred_element_type=jnp.float32)
           + jnp.dot(zd_ref[...], w1b, preferred_element_type=jnp.float32)
           + jnp.dot(d, w1c, preferred_element_type=jnp.float32)
           + b1_ref[...]
           + t2[0:1, :]
           + etf_ref[...] * (t2[1:2, :] - t2[0:1, :]))
    h = jnp.maximum(pre, 0.0)
    logit = jnp.dot(h, w2_ref[...], preferred_element_type=jnp.float32) + b2_ref[...]
    ew_ref[...] = jnp.exp(jax.nn.sigmoid(logit))


def _edge_mlp(ZS, ZDMD, etf, W1, b1, W2, b2, type_emb):
    return pl.pallas_call(
        _edge_mlp_body,
        grid=(ET_TILES,),
        in_specs=[
            pl.BlockSpec((T, D), lambda i: (i, 0)),
            pl.BlockSpec((T, D), lambda i: (i, 0)),
            pl.BlockSpec((T, 1), lambda i: (i, 0)),
            pl.BlockSpec((3 * D + TE, D), lambda i: (0, 0)),
            pl.BlockSpec((1, D), lambda i: (0, 0)),
            pl.BlockSpec((D, 1), lambda i: (0, 0)),
            pl.BlockSpec((1, 1), lambda i: (0, 0)),
            pl.BlockSpec((2, TE), lambda i: (0, 0)),
        ],
        out_specs=pl.BlockSpec((T, 1), lambda i: (i, 0)),
        out_shape=jax.ShapeDtypeStruct((E, 1), jnp.float32),
    )(ZS, ZDMD, etf, W1, b1.reshape(1, D), W2, b2.reshape(1, 1), type_emb)


def _segsum_body(ew_hbm, src_hbm, sump_out, src_v, ew_v, zb_v, shared):
    cid = lax.axis_index("c")
    sid = lax.axis_index("s")
    wid = _wid()

    def zrow(i, carry):
        zb_v[pl.ds(i * 16, 16)] = jnp.zeros((16,), jnp.float32)
        return carry

    lax.fori_loop(0, SL // 16, zrow, 0)
    pltpu.sync_copy(zb_v, shared.at[pl.ds(sid * SL, SL)])
    plsc.subcore_barrier()

    pltpu.sync_copy(src_hbm.at[wid], src_v)
    pltpu.sync_copy(ew_hbm.at[wid], ew_v)

    def body(j, carry):
        pltpu.sync_copy(ew_v.at[j], shared.at[src_v.at[j]], add=True)
        return carry

    lax.fori_loop(0, NCHUNK, body, 0)
    plsc.subcore_barrier()
    pltpu.sync_copy(shared.at[pl.ds(sid * SL, SL)],
                    sump_out.at[cid, pl.ds(sid * SL, SL)])


_sc_segsum = functools.partial(
    pl.kernel,
    _segsum_body,
    out_type=jax.ShapeDtypeStruct((NC, NPAD), jnp.float32),
    mesh=_mesh,
    compiler_params=_sc_params,
    scratch_types=[
        pltpu.VMEM((NCHUNK, CH), jnp.int32),
        pltpu.VMEM((NCHUNK, CH), jnp.float32),
        pltpu.VMEM((SL,), jnp.float32),
        pltpu.VMEM_SHARED((NPAD,), jnp.float32),
    ],
)()


def _sumcomb_body(p0_ref, p1_ref, sum_ref):
    sum_ref[...] = p0_ref[...] + p1_ref[...]


def _sum_combine(p0, p1):
    return pl.pallas_call(
        _sumcomb_body,
        out_shape=jax.ShapeDtypeStruct((NPAD,), jnp.float32),
    )(p0, p1)


def _alpha_body(sum_hbm, ew_hbm, src_hbm, alpha_out, sum_v, src_v, ew_v, al_v):
    wid = _wid()
    pltpu.sync_copy(sum_hbm, sum_v)
    pltpu.sync_copy(src_hbm.at[wid], src_v)
    pltpu.sync_copy(ew_hbm.at[wid], ew_v)

    def body(r, carry):
        for g in range(CH // 16):
            sidx = src_v[r, pl.ds(g * 16, 16)]
            sg = plsc.load_gather(sum_v, [sidx])
            a = ew_v[r, pl.ds(g * 16, 16)] / (sg + 1e-12)
            al_v[r, pl.ds(g * 16, 16)] = a
        return carry

    lax.fori_loop(0, NCHUNK, body, 0)
    pltpu.sync_copy(al_v, alpha_out.at[wid])


_sc_alpha = functools.partial(
    pl.kernel,
    _alpha_body,
    out_type=jax.ShapeDtypeStruct((NW, NCHUNK, CH), jnp.float32),
    mesh=_mesh,
    compiler_params=_sc_params,
    scratch_types=[
        pltpu.VMEM((NPAD,), jnp.float32),
        pltpu.VMEM((NCHUNK, CH), jnp.int32),
        pltpu.VMEM((NCHUNK, CH), jnp.float32),
        pltpu.VMEM((NCHUNK, CH), jnp.float32),
    ],
)()


def _scale_body(alpha_ref, md_ref, msg_ref):
    msg_ref[...] = alpha_ref[...] * md_ref[...]


def _scale_msgs(alpha2d, ZDMD):
    return pl.pallas_call(
        _scale_body,
        grid=(ET_TILES,),
        in_specs=[
            pl.BlockSpec((T, 1), lambda i: (i, 0)),
            pl.BlockSpec((T, D), lambda i: (i, 1)),
        ],
        out_specs=pl.BlockSpec((T, D), lambda i: (i, 0)),
        out_shape=jax.ShapeDtypeStruct((E, D), jnp.float32),
    )(alpha2d, ZDMD)


def _scatter_body(msg_hbm, src_hbm, aggp_out, src_v, mbuf, zb_v, shared):
    cid = lax.axis_index("c")
    sid = lax.axis_index("s")
    wid = _wid()

    def zrow(i, carry):
        for g in range(D // 16):
            zb_v[i, pl.ds(g * 16, 16)] = jnp.zeros((16,), jnp.float32)
        return carry

    lax.fori_loop(0, CH, zrow, 0)
    for q in range(SL // CH):
        pltpu.sync_copy(zb_v, shared.at[pl.ds(sid * SL + q * CH, CH)])
    plsc.subcore_barrier()

    pltpu.sync_copy(src_hbm.at[wid], src_v)
    base = wid * EPW

    def body(j, carry):
        pltpu.sync_copy(msg_hbm.at[pl.ds(base + j * CH, CH)], mbuf)
        pltpu.sync_copy(mbuf, shared.at[src_v.at[j]], add=True)
        return carry

    lax.fori_loop(0, NCHUNK, body, 0)
    plsc.subcore_barrier()
    pltpu.sync_copy(shared.at[pl.ds(sid * SL, SL)],
                    aggp_out.at[cid, pl.ds(sid * SL, SL)])


_sc_scatter = functools.partial(
    pl.kernel,
    _scatter_body,
    out_type=jax.ShapeDtypeStruct((NC, NPAD, D), jnp.float32),
    mesh=_mesh,
    compiler_params=_sc_params,
    scratch_types=[
        pltpu.VMEM((NCHUNK, CH), jnp.int32),
        pltpu.VMEM((CH, D), jnp.float32),
        pltpu.VMEM((CH, D), jnp.float32),
        pltpu.VMEM_SHARED((NPAD, D), jnp.float32),
    ],
)()


def _final_body(z_ref, a0_ref, a1_ref, g_ref, b_ref, out_ref):
    x = z_ref[...] + a0_ref[...] + a1_ref[...]
    mu = jnp.mean(x, axis=-1, keepdims=True)
    xc = x - mu
    var = jnp.mean(xc * xc, axis=-1, keepdims=True)
    out_ref[...] = xc * jax.lax.rsqrt(var + 1e-5) * g_ref[...] + b_ref[...]


def _final_ln(z, agg0, agg1, gamma, beta):
    return pl.pallas_call(
        _final_body,
        out_shape=jax.ShapeDtypeStruct((N, D), jnp.float32),
    )(z, agg0, agg1, gamma.reshape(1, D), beta.reshape(1, D))


def kernel(z, edge_index, edge_type, type_emb, W1, b1, W2, b2,
           Wm1, bm1, Wm2, bm2, gamma, beta):
    src = edge_index[0]
    dst = edge_index[1]
    src3 = src.reshape(NW, NCHUNK, CH)
    dst3 = dst.reshape(NW, NCHUNK, CH)
    etf = edge_type.astype(jnp.float32).reshape(E, 1)

    ZM = _node_mlp(z, Wm1, bm1, Wm2, bm2)
    ZS, ZDMD = _sc_gather(z, ZM, src3, dst3)
    ew2d = _edge_mlp(ZS, ZDMD, etf, W1, b1, W2, b2, type_emb)
    ew3 = ew2d.reshape(NW, NCHUNK, CH)

    SUMP = _sc_segsum(ew3, src3)
    SUM = _sum_combine(SUMP[0], SUMP[1])
    alpha3 = _sc_alpha(SUM, ew3, src3)
    alpha = alpha3.reshape(E)

    MSG = _scale_msgs(alpha.reshape(E, 1), ZDMD)
    AGGP = _sc_scatter(MSG, src3)
    out = _final_ln(z, AGGP[0, :N], AGGP[1, :N], gamma, beta)
    return (out, alpha)


# speedup vs baseline: 8.6738x; 1.4278x over previous
import functools

import jax
import jax.numpy as jnp
from jax import lax
from jax.experimental import pallas as pl
from jax.experimental.pallas import tpu as pltpu
from jax.experimental.pallas import tpu_sc as plsc

N = 10000
E = 320000
D = 128
TE = 8
ET_TILES = 125
T = E // ET_TILES

NC = 2
NS = 16
NW = NC * NS
EPW = E // NW
CH = 80
NCHUNK = EPW // CH
NPAIR = NCHUNK // 2
NG = EPW // 16
NPAD = 10240
SL = NPAD // NS

_mesh = plsc.VectorSubcoreMesh(core_axis_name="c", subcore_axis_name="s",
                               num_cores=NC, num_subcores=NS)
_sc_params = pltpu.CompilerParams(needs_layout_passes=False)


def _wid():
    return lax.axis_index("s") * NC + lax.axis_index("c")


def _node_mlp_body(z_ref, wm1_ref, bm1_ref, wm2_ref, bm2_ref, zm_ref):
    z = z_ref[...]
    h = jnp.maximum(jnp.dot(z, wm1_ref[...], preferred_element_type=jnp.float32)
                    + bm1_ref[...], 0.0)
    m = jnp.dot(h, wm2_ref[...], preferred_element_type=jnp.float32) + bm2_ref[...]
    zm_ref[:, :D] = z
    zm_ref[:, D:] = m


def _node_mlp(z, Wm1, bm1, Wm2, bm2):
    return pl.pallas_call(
        _node_mlp_body,
        out_shape=jax.ShapeDtypeStruct((N, 2 * D), jnp.float32),
    )(z, Wm1, bm1.reshape(1, D), Wm2, bm2.reshape(1, D))


def _gather_body(z_hbm, zm_hbm, src_hbm, dst_hbm, zs_out, zd_out,
                 sidx_v, didx_v, bufa0, bufa1, bufb0, bufb1,
                 sa0, sa1, sb0, sb1):
    wid = _wid()
    pltpu.sync_copy(src_hbm.at[wid], sidx_v)
    pltpu.sync_copy(dst_hbm.at[wid], didx_v)
    base = wid * EPW

    pltpu.async_copy(z_hbm.at[sidx_v.at[0]], bufa0, sa0)
    pltpu.async_copy(zm_hbm.at[didx_v.at[0]], bufb0, sb0)
    pltpu.async_copy(z_hbm.at[sidx_v.at[1]], bufa1, sa1)
    pltpu.async_copy(zm_hbm.at[didx_v.at[1]], bufb1, sb1)

    def body(k, carry):
        j0 = 2 * k
        pltpu.make_async_copy(z_hbm.at[sidx_v.at[j0]], bufa0, sa0).wait()
        pltpu.make_async_copy(zm_hbm.at[didx_v.at[j0]], bufb0, sb0).wait()
        pltpu.sync_copy(bufa0, zs_out.at[pl.ds(base + j0 * CH, CH)])
        pltpu.sync_copy(bufb0, zd_out.at[pl.ds(base + j0 * CH, CH)])

        @pl.when(j0 + 2 < NCHUNK)
        def _():
            pltpu.async_copy(z_hbm.at[sidx_v.at[j0 + 2]], bufa0, sa0)
            pltpu.async_copy(zm_hbm.at[didx_v.at[j0 + 2]], bufb0, sb0)

        pltpu.make_async_copy(z_hbm.at[sidx_v.at[j0 + 1]], bufa1, sa1).wait()
        pltpu.make_async_copy(zm_hbm.at[didx_v.at[j0 + 1]], bufb1, sb1).wait()
        pltpu.sync_copy(bufa1, zs_out.at[pl.ds(base + (j0 + 1) * CH, CH)])
        pltpu.sync_copy(bufb1, zd_out.at[pl.ds(base + (j0 + 1) * CH, CH)])

        @pl.when(j0 + 3 < NCHUNK)
        def _():
            pltpu.async_copy(z_hbm.at[sidx_v.at[j0 + 3]], bufa1, sa1)
            pltpu.async_copy(zm_hbm.at[didx_v.at[j0 + 3]], bufb1, sb1)

        return carry

    lax.fori_loop(0, NPAIR, body, 0)
    jlast = NCHUNK - 1
    pltpu.make_async_copy(z_hbm.at[sidx_v.at[jlast]], bufa0, sa0).wait()
    pltpu.make_async_copy(zm_hbm.at[didx_v.at[jlast]], bufb0, sb0).wait()
    pltpu.sync_copy(bufa0, zs_out.at[pl.ds(base + jlast * CH, CH)])
    pltpu.sync_copy(bufb0, zd_out.at[pl.ds(base + jlast * CH, CH)])


_sc_gather = functools.partial(
    pl.kernel,
    _gather_body,
    out_type=(jax.ShapeDtypeStruct((E, D), jnp.float32),
              jax.ShapeDtypeStruct((E, 2 * D), jnp.float32)),
    mesh=_mesh,
    compiler_params=_sc_params,
    scratch_types=[
        pltpu.VMEM((NCHUNK, CH), jnp.int32),
        pltpu.VMEM((NCHUNK, CH), jnp.int32),
        pltpu.VMEM((CH, D), jnp.float32),
        pltpu.VMEM((CH, D), jnp.float32),
        pltpu.VMEM((CH, 2 * D), jnp.float32),
        pltpu.VMEM((CH, 2 * D), jnp.float32),
        pltpu.SemaphoreType.DMA,
        pltpu.SemaphoreType.DMA,
        pltpu.SemaphoreType.DMA,
        pltpu.SemaphoreType.DMA,
    ],
)()


def _edge_mlp_body(zs_ref, zd_ref, md_ref, etf_ref, w1_ref, b1_ref, w2_ref,
                   b2_ref, temb_ref, ew_ref, msg_ref):
    zs = zs_ref[...]
    d = jnp.abs(zs - zd_ref[...])
    w1a = w1_ref[0:D, :]
    w1b = w1_ref[D:2 * D, :]
    w1c = w1_ref[2 * D:3 * D, :]
    w1d = w1_ref[3 * D:3 * D + TE, :]
    t2 = jnp.dot(temb_ref[...], w1d, preferred_element_type=jnp.float32)
    pre = (jnp.dot(zs, w1a, preferred_element_type=jnp.float32)
           + jnp.dot(zd_ref[...], w1b, preferred_element_type=jnp.float32)
           + jnp.dot(d, w1c, preferred_element_type=jnp.float32)
           + b1_ref[...]
           + t2[0:1, :]
           + etf_ref[...] * (t2[1:2, :] - t2[0:1, :]))
    h = jnp.maximum(pre, 0.0)
    logit = jnp.dot(h, w2_ref[...], preferred_element_type=jnp.float32) + b2_ref[...]
    ew = jnp.exp(jax.nn.sigmoid(logit))
    ew_ref[...] = ew
    msg_ref[...] = ew * md_ref[...]


def _edge_mlp(ZS, ZDMD, etf, W1, b1, W2, b2, type_emb):
    return pl.pallas_call(
        _edge_mlp_body,
        grid=(ET_TILES,),
        in_specs=[
            pl.BlockSpec((T, D), lambda i: (i, 0)),
            pl.BlockSpec((T, D), lambda i: (i, 0)),
            pl.BlockSpec((T, D), lambda i: (i, 1)),
            pl.BlockSpec((T, 1), lambda i: (i, 0)),
            pl.BlockSpec((3 * D + TE, D), lambda i: (0, 0)),
            pl.BlockSpec((1, D), lambda i: (0, 0)),
            pl.BlockSpec((D, 1), lambda i: (0, 0)),
            pl.BlockSpec((1, 1), lambda i: (0, 0)),
            pl.BlockSpec((2, TE), lambda i: (0, 0)),
        ],
        out_specs=[
            pl.BlockSpec((T, 1), lambda i: (i, 0)),
            pl.BlockSpec((T, D), lambda i: (i, 0)),
        ],
        out_shape=[
            jax.ShapeDtypeStruct((E, 1), jnp.float32),
            jax.ShapeDtypeStruct((E, D), jnp.float32),
        ],
    )(ZS, ZDMD, ZDMD, etf, W1, b1.reshape(1, D), W2, b2.reshape(1, 1), type_emb)


def _segsum_body(ew_hbm, src_hbm, sump_out, src_v, ew_v, zb_v, shared):
    cid = lax.axis_index("c")
    sid = lax.axis_index("s")
    wid = _wid()

    def zrow(i, carry):
        zb_v[pl.ds(i * 16, 16)] = jnp.zeros((16,), jnp.float32)
        return carry

    lax.fori_loop(0, SL // 16, zrow, 0)
    pltpu.sync_copy(zb_v, shared.at[pl.ds(sid * SL, SL)])
    plsc.subcore_barrier()

    pltpu.sync_copy(src_hbm.at[wid], src_v)
    pltpu.sync_copy(ew_hbm.at[wid], ew_v)

    def body(j, carry):
        pltpu.sync_copy(ew_v.at[j], shared.at[src_v.at[j]], add=True)
        return carry

    lax.fori_loop(0, NCHUNK, body, 0)
    plsc.subcore_barrier()
    pltpu.sync_copy(shared.at[pl.ds(sid * SL, SL)],
                    sump_out.at[cid, pl.ds(sid * SL, SL)])


_sc_segsum = functools.partial(
    pl.kernel,
    _segsum_body,
    out_type=jax.ShapeDtypeStruct((NC, NPAD), jnp.float32),
    mesh=_mesh,
    compiler_params=_sc_params,
    scratch_types=[
        pltpu.VMEM((NCHUNK, CH), jnp.int32),
        pltpu.VMEM((NCHUNK, CH), jnp.float32),
        pltpu.VMEM((SL,), jnp.float32),
        pltpu.VMEM_SHARED((NPAD,), jnp.float32),
    ],
)()


def _alpha_body(sump_hbm, ew_hbm, src_hbm, alpha_out, s2_v, sum_v, src_v,
                ew_v, al_v):
    wid = _wid()
    pltpu.sync_copy(sump_hbm, s2_v)

    def comb(i, carry):
        sl16 = pl.ds(i * 16, 16)
        sum_v[sl16] = s2_v[0, sl16] + s2_v[1, sl16]
        return carry

    lax.fori_loop(0, NPAD // 16, comb, 0)

    pltpu.sync_copy(src_hbm.at[wid], src_v)
    pltpu.sync_copy(ew_hbm.at[wid], ew_v)

    def body(r, carry):
        for g in range(CH // 16):
            sl16 = pl.ds(g * 16, 16)
            sg = plsc.load_gather(sum_v, [src_v[r, sl16]])
            al_v[r, sl16] = ew_v[r, sl16] / (sg + 1e-12)
        return carry

    lax.fori_loop(0, NCHUNK, body, 0)
    pltpu.sync_copy(al_v, alpha_out.at[wid])


_sc_alpha = functools.partial(
    pl.kernel,
    _alpha_body,
    out_type=jax.ShapeDtypeStruct((NW, NCHUNK, CH), jnp.float32),
    mesh=_mesh,
    compiler_params=_sc_params,
    scratch_types=[
        pltpu.VMEM((NC, NPAD), jnp.float32),
        pltpu.VMEM((NPAD,), jnp.float32),
        pltpu.VMEM((NCHUNK, CH), jnp.int32),
        pltpu.VMEM((NCHUNK, CH), jnp.float32),
        pltpu.VMEM((NCHUNK, CH), jnp.float32),
    ],
)()


def _scatter_body(msg_hbm, src_hbm, aggp_out, src_v, mbuf0, mbuf1, zb_v,
                  shared, sm0, sm1):
    cid = lax.axis_index("c")
    sid = lax.axis_index("s")
    wid = _wid()

    def zrow(i, carry):
        for g in range(D // 16):
            zb_v[i, pl.ds(g * 16, 16)] = jnp.zeros((16,), jnp.float32)
        return carry

    lax.fori_loop(0, CH, zrow, 0)
    for q in range(SL // CH):
        pltpu.sync_copy(zb_v, shared.at[pl.ds(sid * SL + q * CH, CH)])
    plsc.subcore_barrier()

    pltpu.sync_copy(src_hbm.at[wid], src_v)
    base = wid * EPW
    pltpu.async_copy(msg_hbm.at[pl.ds(base, CH)], mbuf0, sm0)
    pltpu.async_copy(msg_hbm.at[pl.ds(base + CH, CH)], mbuf1, sm1)

    def body(k, carry):
        j0 = 2 * k
        pltpu.make_async_copy(msg_hbm.at[pl.ds(base, CH)], mbuf0, sm0).wait()
        pltpu.sync_copy(mbuf0, shared.at[src_v.at[j0]], add=True)

        @pl.when(j0 + 2 < NCHUNK)
        def _():
            pltpu.async_copy(msg_hbm.at[pl.ds(base + (j0 + 2) * CH, CH)],
                             mbuf0, sm0)

        pltpu.make_async_copy(msg_hbm.at[pl.ds(base, CH)], mbuf1, sm1).wait()
        pltpu.sync_copy(mbuf1, shared.at[src_v.at[j0 + 1]], add=True)

        @pl.when(j0 + 3 < NCHUNK)
        def _():
            pltpu.async_copy(msg_hbm.at[pl.ds(base + (j0 + 3) * CH, CH)],
                             mbuf1, sm1)

        return carry

    lax.fori_loop(0, NPAIR, body, 0)
    jlast = NCHUNK - 1
    pltpu.make_async_copy(msg_hbm.at[pl.ds(base, CH)], mbuf0, sm0).wait()
    pltpu.sync_copy(mbuf0, shared.at[src_v.at[jlast]], add=True)
    plsc.subcore_barrier()
    pltpu.sync_copy(shared.at[pl.ds(sid * SL, SL)],
                    aggp_out.at[cid, pl.ds(sid * SL, SL)])


_sc_scatter = functools.partial(
    pl.kernel,
    _scatter_body,
    out_type=jax.ShapeDtypeStruct((NC, NPAD, D), jnp.float32),
    mesh=_mesh,
    compiler_params=_sc_params,
    scratch_types=[
        pltpu.VMEM((NCHUNK, CH), jnp.int32),
        pltpu.VMEM((CH, D), jnp.float32),
        pltpu.VMEM((CH, D), jnp.float32),
        pltpu.VMEM((CH, D), jnp.float32),
        pltpu.VMEM_SHARED((NPAD, D), jnp.float32),
        pltpu.SemaphoreType.DMA,
        pltpu.SemaphoreType.DMA,
    ],
)()


def _final_body(z_ref, u0_ref, u1_ref, s0_ref, s1_ref, g_ref, b_ref, out_ref):
    agg = (u0_ref[...] + u1_ref[...]) / (s0_ref[...] + s1_ref[...] + 1e-12)
    x = z_ref[...] + agg
    mu = jnp.mean(x, axis=-1, keepdims=True)
    xc = x - mu
    var = jnp.mean(xc * xc, axis=-1, keepdims=True)
    out_ref[...] = xc * jax.lax.rsqrt(var + 1e-5) * g_ref[...] + b_ref[...]


def _final_ln(z, agg0, agg1, s0, s1, gamma, beta):
    return pl.pallas_call(
        _final_body,
        out_shape=jax.ShapeDtypeStruct((N, D), jnp.float32),
    )(z, agg0, agg1, s0, s1, gamma.reshape(1, D), beta.reshape(1, D))


def kernel(z, edge_index, edge_type, type_emb, W1, b1, W2, b2,
           Wm1, bm1, Wm2, bm2, gamma, beta):
    src = edge_index[0]
    dst = edge_index[1]
    src3 = src.reshape(NW, NCHUNK, CH)
    dst3 = dst.reshape(NW, NCHUNK, CH)
    etf = edge_type.astype(jnp.float32).reshape(E, 1)

    ZM = _node_mlp(z, Wm1, bm1, Wm2, bm2)
    ZS, ZDMD = _sc_gather(z, ZM, src3, dst3)
    ew2d, MSG = _edge_mlp(ZS, ZDMD, etf, W1, b1, W2, b2, type_emb)
    ew3 = ew2d.reshape(NW, NCHUNK, CH)

    SUMP = _sc_segsum(ew3, src3)
    alpha3 = _sc_alpha(SUMP, ew3, src3)
    alpha = alpha3.reshape(E)

    AGGP = _sc_scatter(MSG, src3)
    out = _final_ln(z, AGGP[0, :N], AGGP[1, :N],
                    SUMP[0, :N].reshape(N, 1), SUMP[1, :N].reshape(N, 1),
                    gamma, beta)
    return (out, alpha)


# speedup vs baseline: 10.1649x; 1.1719x over previous
import functools

import jax
import jax.numpy as jnp
from jax import lax
from jax.experimental import pallas as pl
from jax.experimental.pallas import tpu as pltpu
from jax.experimental.pallas import tpu_sc as plsc

N = 10000
E = 320000
D = 128
TE = 8
ET_TILES = 125
T = E // ET_TILES

NC = 2
NS = 16
NW = NC * NS
EPW = E // NW
CH = 80
NCHUNK = EPW // CH
NPAIR = NCHUNK // 2
NG = EPW // 16
NPAD = 10240
SL = NPAD // NS

_mesh = plsc.VectorSubcoreMesh(core_axis_name="c", subcore_axis_name="s",
                               num_cores=NC, num_subcores=NS)
_sc_params = pltpu.CompilerParams(needs_layout_passes=False)


def _wid():
    return lax.axis_index("s") * NC + lax.axis_index("c")


def _node_mlp_body(z_ref, wm1_ref, bm1_ref, wm2_ref, bm2_ref, zmp_ref):
    z = z_ref[...]
    h = jnp.maximum(jnp.dot(z, wm1_ref[...], preferred_element_type=jnp.float32)
                    + bm1_ref[...], 0.0)
    m = jnp.dot(h, wm2_ref[...], preferred_element_type=jnp.float32) + bm2_ref[...]
    zu = lax.bitcast_convert_type(z.astype(jnp.bfloat16), jnp.uint16)
    mu = lax.bitcast_convert_type(m.astype(jnp.bfloat16), jnp.uint16)
    packed = zu.astype(jnp.uint32) | (mu.astype(jnp.uint32) << 16)
    zmp_ref[...] = lax.bitcast_convert_type(packed, jnp.int32)


def _node_mlp(z, Wm1, bm1, Wm2, bm2):
    return pl.pallas_call(
        _node_mlp_body,
        out_shape=jax.ShapeDtypeStruct((N, D), jnp.int32),
    )(z, Wm1, bm1.reshape(1, D), Wm2, bm2.reshape(1, D))


def _gather_body(z_hbm, zmp_hbm, src_hbm, dst_hbm, zs_out, zdp_out,
                 sidx_v, didx_v,
                 bs0, bs1, bs2, bs3, bd0, bd1, bd2, bd3,
                 gs0, gs1, gs2, gs3, gd0, gd1, gd2, gd3,
                 ws0, ws1, ws2, ws3, wd0, wd1, wd2, wd3):
    wid = _wid()
    pltpu.sync_copy(src_hbm.at[wid], sidx_v)
    pltpu.sync_copy(dst_hbm.at[wid], didx_v)
    base = wid * EPW
    BS = [bs0, bs1, bs2, bs3]
    BD = [bd0, bd1, bd2, bd3]
    GS = [gs0, gs1, gs2, gs3]
    GD = [gd0, gd1, gd2, gd3]
    WS = [ws0, ws1, ws2, ws3]
    WD = [wd0, wd1, wd2, wd3]

    def issue_g(i, j):
        pltpu.async_copy(z_hbm.at[sidx_v.at[j]], BS[i], GS[i])
        pltpu.async_copy(zmp_hbm.at[didx_v.at[j]], BD[i], GD[i])

    def wait_g(i, j):
        pltpu.make_async_copy(z_hbm.at[sidx_v.at[j]], BS[i], GS[i]).wait()
        pltpu.make_async_copy(zmp_hbm.at[didx_v.at[j]], BD[i], GD[i]).wait()

    def issue_w(i, j):
        row = pl.ds(base + j * CH, CH)
        pltpu.async_copy(BS[i], zs_out.at[row], WS[i])
        pltpu.async_copy(BD[i], zdp_out.at[row], WD[i])

    def wait_w(i):
        pltpu.make_async_copy(z_hbm.at[pl.ds(0, CH)], BS[i], WS[i]).wait()
        pltpu.make_async_copy(z_hbm.at[pl.ds(0, CH)], BD[i], WD[i]).wait()

    issue_g(0, 0)
    issue_g(1, 1)
    wait_g(0, 0)
    issue_w(0, 0)
    issue_g(2, 2)
    wait_g(1, 1)
    issue_w(1, 1)
    issue_g(3, 3)
    wait_g(2, 2)
    issue_w(2, 2)
    wait_w(0)
    issue_g(0, 4)
    wait_g(3, 3)
    issue_w(3, 3)
    wait_w(1)
    issue_g(1, 5)

    def body(k, carry):
        for i in range(4):
            j = 4 * k + i
            wait_g(i, j)
            issue_w(i, j)

            @pl.when(j + 2 < NCHUNK)
            def _():
                li = (i + 2) % 4
                wait_w(li)
                issue_g(li, j + 2)

        return carry

    lax.fori_loop(1, (NCHUNK - 1) // 4, body, 0)
    jlast = NCHUNK - 1
    wait_g(0, jlast)
    issue_w(0, jlast)
    wait_w(1)
    wait_w(2)
    wait_w(3)
    wait_w(0)


_sc_gather = functools.partial(
    pl.kernel,
    _gather_body,
    out_type=(jax.ShapeDtypeStruct((E, D), jnp.float32),
              jax.ShapeDtypeStruct((E, D), jnp.int32)),
    mesh=_mesh,
    compiler_params=_sc_params,
    scratch_types=[
        pltpu.VMEM((NCHUNK, CH), jnp.int32),
        pltpu.VMEM((NCHUNK, CH), jnp.int32),
        pltpu.VMEM((CH, D), jnp.float32),
        pltpu.VMEM((CH, D), jnp.float32),
        pltpu.VMEM((CH, D), jnp.float32),
        pltpu.VMEM((CH, D), jnp.float32),
        pltpu.VMEM((CH, D), jnp.int32),
        pltpu.VMEM((CH, D), jnp.int32),
        pltpu.VMEM((CH, D), jnp.int32),
        pltpu.VMEM((CH, D), jnp.int32),
    ] + [pltpu.SemaphoreType.DMA] * 16,
)()


def _edge_mlp_body(zs_ref, zdp_ref, etf_ref, w1_ref, b1_ref, w2_ref,
                   b2_ref, temb_ref, ew_ref, msg_ref):
    zs = zs_ref[...]
    word = lax.bitcast_convert_type(zdp_ref[...], jnp.uint32)
    zd = lax.bitcast_convert_type(word.astype(jnp.uint16), jnp.bfloat16)
    md = lax.bitcast_convert_type((word >> 16).astype(jnp.uint16), jnp.bfloat16)
    zdf = zd.astype(jnp.float32)
    d = jnp.abs(zs - zdf).astype(jnp.bfloat16)
    w1a = w1_ref[0:D, :].astype(jnp.bfloat16)
    w1b = w1_ref[D:2 * D, :].astype(jnp.bfloat16)
    w1c = w1_ref[2 * D:3 * D, :].astype(jnp.bfloat16)
    w1d = w1_ref[3 * D:3 * D + TE, :]
    t2 = jnp.dot(temb_ref[...], w1d, preferred_element_type=jnp.float32)
    pre = (jnp.dot(zs.astype(jnp.bfloat16), w1a, preferred_element_type=jnp.float32)
           + jnp.dot(zd, w1b, preferred_element_type=jnp.float32)
           + jnp.dot(d, w1c, preferred_element_type=jnp.float32)
           + b1_ref[...]
           + t2[0:1, :]
           + etf_ref[...] * (t2[1:2, :] - t2[0:1, :]))
    h = jnp.maximum(pre, 0.0)
    logit = jnp.dot(h, w2_ref[...], preferred_element_type=jnp.float32) + b2_ref[...]
    ew = jnp.exp(jax.nn.sigmoid(logit))
    ew_ref[...] = ew
    msg_ref[...] = ew * md.astype(jnp.float32)


def _edge_mlp(ZS, ZDP, etf, W1, b1, W2, b2, type_emb):
    return pl.pallas_call(
        _edge_mlp_body,
        grid=(ET_TILES,),
        in_specs=[
            pl.BlockSpec((T, D), lambda i: (i, 0)),
            pl.BlockSpec((T, D), lambda i: (i, 0)),
            pl.BlockSpec((T, 1), lambda i: (i, 0)),
            pl.BlockSpec((3 * D + TE, D), lambda i: (0, 0)),
            pl.BlockSpec((1, D), lambda i: (0, 0)),
            pl.BlockSpec((D, 1), lambda i: (0, 0)),
            pl.BlockSpec((1, 1), lambda i: (0, 0)),
            pl.BlockSpec((2, TE), lambda i: (0, 0)),
        ],
        out_specs=[
            pl.BlockSpec((T, 1), lambda i: (i, 0)),
            pl.BlockSpec((T, D), lambda i: (i, 0)),
        ],
        out_shape=[
            jax.ShapeDtypeStruct((E, 1), jnp.float32),
            jax.ShapeDtypeStruct((E, D), jnp.float32),
        ],
    )(ZS, ZDP, etf, W1, b1.reshape(1, D), W2, b2.reshape(1, 1), type_emb)


def _segsum_body(ew_hbm, src_hbm, sump_out, src_v, ew_v, zb_v, shared):
    cid = lax.axis_index("c")
    sid = lax.axis_index("s")
    wid = _wid()

    def zrow(i, carry):
        zb_v[pl.ds(i * 16, 16)] = jnp.zeros((16,), jnp.float32)
        return carry

    lax.fori_loop(0, SL // 16, zrow, 0)
    pltpu.sync_copy(zb_v, shared.at[pl.ds(sid * SL, SL)])
    plsc.subcore_barrier()

    pltpu.sync_copy(src_hbm.at[wid], src_v)
    pltpu.sync_copy(ew_hbm.at[wid], ew_v)

    def body(j, carry):
        pltpu.sync_copy(ew_v.at[j], shared.at[src_v.at[j]], add=True)
        return carry

    lax.fori_loop(0, NCHUNK, body, 0)
    plsc.subcore_barrier()
    pltpu.sync_copy(shared.at[pl.ds(sid * SL, SL)],
                    sump_out.at[cid, pl.ds(sid * SL, SL)])


_sc_segsum = functools.partial(
    pl.kernel,
    _segsum_body,
    out_type=jax.ShapeDtypeStruct((NC, NPAD), jnp.float32),
    mesh=_mesh,
    compiler_params=_sc_params,
    scratch_types=[
        pltpu.VMEM((NCHUNK, CH), jnp.int32),
        pltpu.VMEM((NCHUNK, CH), jnp.float32),
        pltpu.VMEM((SL,), jnp.float32),
        pltpu.VMEM_SHARED((NPAD,), jnp.float32),
    ],
)()


def _alpha_body(sump_hbm, ew_hbm, src_hbm, alpha_out, s2_v, sum_v, src_v,
                ew_v, al_v):
    wid = _wid()
    pltpu.sync_copy(sump_hbm, s2_v)

    def comb(i, carry):
        sl16 = pl.ds(i * 16, 16)
        sum_v[sl16] = s2_v[0, sl16] + s2_v[1, sl16]
        return carry

    lax.fori_loop(0, NPAD // 16, comb, 0)

    pltpu.sync_copy(src_hbm.at[wid], src_v)
    pltpu.sync_copy(ew_hbm.at[wid], ew_v)

    def body(r, carry):
        for g in range(CH // 16):
            sl16 = pl.ds(g * 16, 16)
            sg = plsc.load_gather(sum_v, [src_v[r, sl16]])
            al_v[r, sl16] = ew_v[r, sl16] / (sg + 1e-12)
        return carry

    lax.fori_loop(0, NCHUNK, body, 0)
    pltpu.sync_copy(al_v, alpha_out.at[wid])


_sc_alpha = functools.partial(
    pl.kernel,
    _alpha_body,
    out_type=jax.ShapeDtypeStruct((NW, NCHUNK, CH), jnp.float32),
    mesh=_mesh,
    compiler_params=_sc_params,
    scratch_types=[
        pltpu.VMEM((NC, NPAD), jnp.float32),
        pltpu.VMEM((NPAD,), jnp.float32),
        pltpu.VMEM((NCHUNK, CH), jnp.int32),
        pltpu.VMEM((NCHUNK, CH), jnp.float32),
        pltpu.VMEM((NCHUNK, CH), jnp.float32),
    ],
)()


def _scatter_body(msg_hbm, src_hbm, aggp_out, src_v, mbuf0, mbuf1, zb_v,
                  shared, sm0, sm1):
    cid = lax.axis_index("c")
    sid = lax.axis_index("s")
    wid = _wid()

    def zrow(i, carry):
        for g in range(D // 16):
            zb_v[i, pl.ds(g * 16, 16)] = jnp.zeros((16,), jnp.float32)
        return carry

    lax.fori_loop(0, CH, zrow, 0)
    for q in range(SL // CH):
        pltpu.sync_copy(zb_v, shared.at[pl.ds(sid * SL + q * CH, CH)])
    plsc.subcore_barrier()

    pltpu.sync_copy(src_hbm.at[wid], src_v)
    base = wid * EPW
    pltpu.async_copy(msg_hbm.at[pl.ds(base, CH)], mbuf0, sm0)
    pltpu.async_copy(msg_hbm.at[pl.ds(base + CH, CH)], mbuf1, sm1)

    def body(k, carry):
        j0 = 2 * k
        pltpu.make_async_copy(msg_hbm.at[pl.ds(base, CH)], mbuf0, sm0).wait()
        pltpu.sync_copy(mbuf0, shared.at[src_v.at[j0]], add=True)

        @pl.when(j0 + 2 < NCHUNK)
        def _():
            pltpu.async_copy(msg_hbm.at[pl.ds(base + (j0 + 2) * CH, CH)],
                             mbuf0, sm0)

        pltpu.make_async_copy(msg_hbm.at[pl.ds(base, CH)], mbuf1, sm1).wait()
        pltpu.sync_copy(mbuf1, shared.at[src_v.at[j0 + 1]], add=True)

        @pl.when(j0 + 3 < NCHUNK)
        def _():
            pltpu.async_copy(msg_hbm.at[pl.ds(base + (j0 + 3) * CH, CH)],
                             mbuf1, sm1)

        return carry

    lax.fori_loop(0, NPAIR, body, 0)
    jlast = NCHUNK - 1
    pltpu.make_async_copy(msg_hbm.at[pl.ds(base, CH)], mbuf0, sm0).wait()
    pltpu.sync_copy(mbuf0, shared.at[src_v.at[jlast]], add=True)
    plsc.subcore_barrier()
    pltpu.sync_copy(shared.at[pl.ds(sid * SL, SL)],
                    aggp_out.at[cid, pl.ds(sid * SL, SL)])


_sc_scatter = functools.partial(
    pl.kernel,
    _scatter_body,
    out_type=jax.ShapeDtypeStruct((NC, NPAD, D), jnp.float32),
    mesh=_mesh,
    compiler_params=_sc_params,
    scratch_types=[
        pltpu.VMEM((NCHUNK, CH), jnp.int32),
        pltpu.VMEM((CH, D), jnp.float32),
        pltpu.VMEM((CH, D), jnp.float32),
        pltpu.VMEM((CH, D), jnp.float32),
        pltpu.VMEM_SHARED((NPAD, D), jnp.float32),
        pltpu.SemaphoreType.DMA,
        pltpu.SemaphoreType.DMA,
    ],
)()


def _final_body(z_ref, u0_ref, u1_ref, s0_ref, s1_ref, g_ref, b_ref, out_ref):
    agg = (u0_ref[...] + u1_ref[...]) / (s0_ref[...] + s1_ref[...] + 1e-12)
    x = z_ref[...] + agg
    mu = jnp.mean(x, axis=-1, keepdims=True)
    xc = x - mu
    var = jnp.mean(xc * xc, axis=-1, keepdims=True)
    out_ref[...] = xc * jax.lax.rsqrt(var + 1e-5) * g_ref[...] + b_ref[...]


def _final_ln(z, agg0, agg1, s0, s1, gamma, beta):
    return pl.pallas_call(
        _final_body,
        out_shape=jax.ShapeDtypeStruct((N, D), jnp.float32),
    )(z, agg0, agg1, s0, s1, gamma.reshape(1, D), beta.reshape(1, D))


def kernel(z, edge_index, edge_type, type_emb, W1, b1, W2, b2,
           Wm1, bm1, Wm2, bm2, gamma, beta):
    src = edge_index[0]
    dst = edge_index[1]
    src3 = src.reshape(NW, NCHUNK, CH)
    dst3 = dst.reshape(NW, NCHUNK, CH)
    etf = edge_type.astype(jnp.float32).reshape(E, 1)

    ZMP = _node_mlp(z, Wm1, bm1, Wm2, bm2)
    ZS, ZDP = _sc_gather(z, ZMP, src3, dst3)
    ew2d, MSG = _edge_mlp(ZS, ZDP, etf, W1, b1, W2, b2, type_emb)
    ew3 = ew2d.reshape(NW, NCHUNK, CH)

    SUMP = _sc_segsum(ew3, src3)
    alpha3 = _sc_alpha(SUMP, ew3, src3)
    alpha = alpha3.reshape(E)

    AGGP = _sc_scatter(MSG, src3)
    out = _final_ln(z, AGGP[0, :N], AGGP[1, :N],
                    SUMP[0, :N].reshape(N, 1), SUMP[1, :N].reshape(N, 1),
                    gamma, beta)
    return (out, alpha)


# speedup vs baseline: 10.3062x; 1.0139x over previous
import functools

import jax
import jax.numpy as jnp
from jax import lax
from jax.experimental import pallas as pl
from jax.experimental.pallas import tpu as pltpu
from jax.experimental.pallas import tpu_sc as plsc

N = 10000
E = 320000
D = 128
TE = 8
ET_TILES = 125
T = E // ET_TILES

NC = 2
NS = 16
NW = NC * NS
EPW = E // NW
CH = 80
NCHUNK = EPW // CH
NPAIR = NCHUNK // 2
NG = EPW // 16
NPAD = 10240
SL = NPAD // NS

_mesh = plsc.VectorSubcoreMesh(core_axis_name="c", subcore_axis_name="s",
                               num_cores=NC, num_subcores=NS)
_sc_params = pltpu.CompilerParams(needs_layout_passes=False)


def _wid():
    return lax.axis_index("s") * NC + lax.axis_index("c")


def _node_mlp_body(z_ref, wm1_ref, bm1_ref, wm2_ref, bm2_ref, zmp_ref):
    z = z_ref[...]
    h = jnp.maximum(jnp.dot(z, wm1_ref[...], preferred_element_type=jnp.float32)
                    + bm1_ref[...], 0.0)
    m = jnp.dot(h, wm2_ref[...], preferred_element_type=jnp.float32) + bm2_ref[...]
    zu = lax.bitcast_convert_type(z.astype(jnp.bfloat16), jnp.uint16)
    mu = lax.bitcast_convert_type(m.astype(jnp.bfloat16), jnp.uint16)
    packed = zu.astype(jnp.uint32) | (mu.astype(jnp.uint32) << 16)
    zmp_ref[...] = lax.bitcast_convert_type(packed, jnp.int32)


def _node_mlp(z, Wm1, bm1, Wm2, bm2):
    return pl.pallas_call(
        _node_mlp_body,
        out_shape=jax.ShapeDtypeStruct((N, D), jnp.int32),
    )(z, Wm1, bm1.reshape(1, D), Wm2, bm2.reshape(1, D))


def _gather_body(z_hbm, zmp_hbm, src_hbm, dst_hbm, zs_out, zdp_out,
                 sidx_v, didx_v,
                 bs0, bs1, bs2, bs3, bd0, bd1, bd2, bd3,
                 gs0, gs1, gs2, gs3, gd0, gd1, gd2, gd3,
                 ws0, ws1, ws2, ws3, wd0, wd1, wd2, wd3):
    wid = _wid()
    pltpu.sync_copy(src_hbm.at[wid], sidx_v)
    pltpu.sync_copy(dst_hbm.at[wid], didx_v)
    base = wid * EPW
    BS = [bs0, bs1, bs2, bs3]
    BD = [bd0, bd1, bd2, bd3]
    GS = [gs0, gs1, gs2, gs3]
    GD = [gd0, gd1, gd2, gd3]
    WS = [ws0, ws1, ws2, ws3]
    WD = [wd0, wd1, wd2, wd3]

    def issue_g(i, j):
        pltpu.async_copy(z_hbm.at[sidx_v.at[j]], BS[i], GS[i])
        pltpu.async_copy(zmp_hbm.at[didx_v.at[j]], BD[i], GD[i])

    def wait_g(i, j):
        pltpu.make_async_copy(z_hbm.at[sidx_v.at[j]], BS[i], GS[i]).wait()
        pltpu.make_async_copy(zmp_hbm.at[didx_v.at[j]], BD[i], GD[i]).wait()

    def issue_w(i, j):
        row = pl.ds(base + j * CH, CH)
        pltpu.async_copy(BS[i], zs_out.at[row], WS[i])
        pltpu.async_copy(BD[i], zdp_out.at[row], WD[i])

    def wait_w(i):
        pltpu.make_async_copy(z_hbm.at[pl.ds(0, CH)], BS[i], WS[i]).wait()
        pltpu.make_async_copy(z_hbm.at[pl.ds(0, CH)], BD[i], WD[i]).wait()

    issue_g(0, 0)
    issue_g(1, 1)
    wait_g(0, 0)
    issue_w(0, 0)
    issue_g(2, 2)
    wait_g(1, 1)
    issue_w(1, 1)
    issue_g(3, 3)
    wait_g(2, 2)
    issue_w(2, 2)
    wait_w(0)
    issue_g(0, 4)
    wait_g(3, 3)
    issue_w(3, 3)
    wait_w(1)
    issue_g(1, 5)

    def body(k, carry):
        for i in range(4):
            j = 4 * k + i
            wait_g(i, j)
            issue_w(i, j)

            @pl.when(j + 2 < NCHUNK)
            def _():
                li = (i + 2) % 4
                wait_w(li)
                issue_g(li, j + 2)

        return carry

    lax.fori_loop(1, (NCHUNK - 1) // 4, body, 0)
    jlast = NCHUNK - 1
    wait_g(0, jlast)
    issue_w(0, jlast)
    wait_w(1)
    wait_w(2)
    wait_w(3)
    wait_w(0)


_sc_gather = functools.partial(
    pl.kernel,
    _gather_body,
    out_type=(jax.ShapeDtypeStruct((E, D), jnp.float32),
              jax.ShapeDtypeStruct((E, D), jnp.int32)),
    mesh=_mesh,
    compiler_params=_sc_params,
    scratch_types=[
        pltpu.VMEM((NCHUNK, CH), jnp.int32),
        pltpu.VMEM((NCHUNK, CH), jnp.int32),
        pltpu.VMEM((CH, D), jnp.float32),
        pltpu.VMEM((CH, D), jnp.float32),
        pltpu.VMEM((CH, D), jnp.float32),
        pltpu.VMEM((CH, D), jnp.float32),
        pltpu.VMEM((CH, D), jnp.int32),
        pltpu.VMEM((CH, D), jnp.int32),
        pltpu.VMEM((CH, D), jnp.int32),
        pltpu.VMEM((CH, D), jnp.int32),
    ] + [pltpu.SemaphoreType.DMA] * 16,
)()


def _edge_mlp_body(zs_ref, zdp_ref, etf_ref, w1_ref, b1_ref, w2_ref,
                   b2_ref, temb_ref, ew_ref, msg_ref):
    zs = zs_ref[...]
    word = lax.bitcast_convert_type(zdp_ref[...], jnp.uint32)
    zd = lax.bitcast_convert_type(word.astype(jnp.uint16), jnp.bfloat16)
    md = lax.bitcast_convert_type((word >> 16).astype(jnp.uint16), jnp.bfloat16)
    zdf = zd.astype(jnp.float32)
    d = jnp.abs(zs - zdf).astype(jnp.bfloat16)
    w1a = w1_ref[0:D, :].astype(jnp.bfloat16)
    w1b = w1_ref[D:2 * D, :].astype(jnp.bfloat16)
    w1c = w1_ref[2 * D:3 * D, :].astype(jnp.bfloat16)
    w1d = w1_ref[3 * D:3 * D + TE, :]
    t2 = jnp.dot(temb_ref[...], w1d, preferred_element_type=jnp.float32)
    pre = (jnp.dot(zs.astype(jnp.bfloat16), w1a, preferred_element_type=jnp.float32)
           + jnp.dot(zd, w1b, preferred_element_type=jnp.float32)
           + jnp.dot(d, w1c, preferred_element_type=jnp.float32)
           + b1_ref[...]
           + t2[0:1, :]
           + etf_ref[...] * (t2[1:2, :] - t2[0:1, :]))
    h = jnp.maximum(pre, 0.0)
    logit = jnp.dot(h, w2_ref[...], preferred_element_type=jnp.float32) + b2_ref[...]
    ew = jnp.exp(jax.nn.sigmoid(logit))
    ew_ref[...] = ew
    msg_ref[...] = ew * md.astype(jnp.float32)


def _edge_mlp(ZS, ZDP, etf, W1, b1, W2, b2, type_emb):
    return pl.pallas_call(
        _edge_mlp_body,
        grid=(ET_TILES,),
        in_specs=[
            pl.BlockSpec((T, D), lambda i: (i, 0)),
            pl.BlockSpec((T, D), lambda i: (i, 0)),
            pl.BlockSpec((T, 1), lambda i: (i, 0)),
            pl.BlockSpec((3 * D + TE, D), lambda i: (0, 0)),
            pl.BlockSpec((1, D), lambda i: (0, 0)),
            pl.BlockSpec((D, 1), lambda i: (0, 0)),
            pl.BlockSpec((1, 1), lambda i: (0, 0)),
            pl.BlockSpec((2, TE), lambda i: (0, 0)),
        ],
        out_specs=[
            pl.BlockSpec((T, 1), lambda i: (i, 0)),
            pl.BlockSpec((T, D), lambda i: (i, 0)),
        ],
        out_shape=[
            jax.ShapeDtypeStruct((E, 1), jnp.float32),
            jax.ShapeDtypeStruct((E, D), jnp.float32),
        ],
    )(ZS, ZDP, etf, W1, b1.reshape(1, D), W2, b2.reshape(1, 1), type_emb)


def _alpha_body(sump_hbm, ew_hbm, src_hbm, alpha_out, s2_v, sum_v, src_v,
                ew_v, al_v):
    wid = _wid()
    pltpu.sync_copy(sump_hbm, s2_v)

    def comb(i, carry):
        sl16 = pl.ds(i * 16, 16)
        sum_v[sl16] = s2_v[0, sl16] + s2_v[1, sl16]
        return carry

    lax.fori_loop(0, NPAD // 16, comb, 0)

    pltpu.sync_copy(src_hbm.at[wid], src_v)
    pltpu.sync_copy(ew_hbm.at[wid], ew_v)

    def body(r, carry):
        for g in range(CH // 16):
            sl16 = pl.ds(g * 16, 16)
            sg = plsc.load_gather(sum_v, [src_v[r, sl16]])
            al_v[r, sl16] = ew_v[r, sl16] / (sg + 1e-12)
        return carry

    lax.fori_loop(0, NCHUNK, body, 0)
    pltpu.sync_copy(al_v, alpha_out.at[wid])


_sc_alpha = functools.partial(
    pl.kernel,
    _alpha_body,
    out_type=jax.ShapeDtypeStruct((NW, NCHUNK, CH), jnp.float32),
    mesh=_mesh,
    compiler_params=_sc_params,
    scratch_types=[
        pltpu.VMEM((NC, NPAD), jnp.float32),
        pltpu.VMEM((NPAD,), jnp.float32),
        pltpu.VMEM((NCHUNK, CH), jnp.int32),
        pltpu.VMEM((NCHUNK, CH), jnp.float32),
        pltpu.VMEM((NCHUNK, CH), jnp.float32),
    ],
)()


def _scatter_body(msg_hbm, ew_hbm, src_hbm, aggp_out, sump_out,
                  src_v, mbuf0, mbuf1, eb0, eb1, zb_v, shared, sumsh,
                  sm0, sm1, se0, se1):
    cid = lax.axis_index("c")
    sid = lax.axis_index("s")
    wid = _wid()

    def zrow(i, carry):
        for g in range(D // 16):
            zb_v[i, pl.ds(g * 16, 16)] = jnp.zeros((16,), jnp.float32)
        return carry

    lax.fori_loop(0, CH, zrow, 0)
    for q in range(SL // CH):
        pltpu.sync_copy(zb_v, shared.at[pl.ds(sid * SL + q * CH, CH)])
    for q in range(SL // D):
        pltpu.sync_copy(zb_v.at[0], sumsh.at[pl.ds(sid * SL + q * D, D)])
    plsc.subcore_barrier()

    pltpu.sync_copy(src_hbm.at[wid], src_v)
    base = wid * EPW
    pltpu.async_copy(msg_hbm.at[pl.ds(base, CH)], mbuf0, sm0)
    pltpu.async_copy(ew_hbm.at[pl.ds(base, CH)], eb0, se0)
    pltpu.async_copy(msg_hbm.at[pl.ds(base + CH, CH)], mbuf1, sm1)
    pltpu.async_copy(ew_hbm.at[pl.ds(base + CH, CH)], eb1, se1)

    def step(j, mbuf, eb, sm, se):
        pltpu.make_async_copy(msg_hbm.at[pl.ds(base, CH)], mbuf, sm).wait()
        pltpu.make_async_copy(ew_hbm.at[pl.ds(base, CH)], eb, se).wait()
        pltpu.sync_copy(mbuf, shared.at[src_v.at[j]], add=True)
        pltpu.sync_copy(eb, sumsh.at[src_v.at[j]], add=True)

        @pl.when(j + 2 < NCHUNK)
        def _():
            pltpu.async_copy(msg_hbm.at[pl.ds(base + (j + 2) * CH, CH)],
                             mbuf, sm)
            pltpu.async_copy(ew_hbm.at[pl.ds(base + (j + 2) * CH, CH)],
                             eb, se)

    def body(k, carry):
        j0 = 2 * k
        step(j0, mbuf0, eb0, sm0, se0)
        step(j0 + 1, mbuf1, eb1, sm1, se1)
        return carry

    lax.fori_loop(0, NPAIR, body, 0)
    step(NCHUNK - 1, mbuf0, eb0, sm0, se0)
    plsc.subcore_barrier()
    pltpu.sync_copy(shared.at[pl.ds(sid * SL, SL)],
                    aggp_out.at[cid, pl.ds(sid * SL, SL)])
    pltpu.sync_copy(sumsh.at[pl.ds(sid * SL, SL)],
                    sump_out.at[cid, pl.ds(sid * SL, SL)])


_sc_scatter = functools.partial(
    pl.kernel,
    _scatter_body,
    out_type=(jax.ShapeDtypeStruct((NC, NPAD, D), jnp.float32),
              jax.ShapeDtypeStruct((NC, NPAD), jnp.float32)),
    mesh=_mesh,
    compiler_params=_sc_params,
    scratch_types=[
        pltpu.VMEM((NCHUNK, CH), jnp.int32),
        pltpu.VMEM((CH, D), jnp.float32),
        pltpu.VMEM((CH, D), jnp.float32),
        pltpu.VMEM((CH,), jnp.float32),
        pltpu.VMEM((CH,), jnp.float32),
        pltpu.VMEM((CH, D), jnp.float32),
        pltpu.VMEM_SHARED((NPAD, D), jnp.float32),
        pltpu.VMEM_SHARED((NPAD,), jnp.float32),
        pltpu.SemaphoreType.DMA,
        pltpu.SemaphoreType.DMA,
        pltpu.SemaphoreType.DMA,
        pltpu.SemaphoreType.DMA,
    ],
)()


def _final_body(z_ref, u0_ref, u1_ref, s0_ref, s1_ref, g_ref, b_ref, out_ref):
    agg = (u0_ref[...] + u1_ref[...]) / (s0_ref[...] + s1_ref[...] + 1e-12)
    x = z_ref[...] + agg
    mu = jnp.mean(x, axis=-1, keepdims=True)
    xc = x - mu
    var = jnp.mean(xc * xc, axis=-1, keepdims=True)
    out_ref[...] = xc * jax.lax.rsqrt(var + 1e-5) * g_ref[...] + b_ref[...]


def _final_ln(z, agg0, agg1, s0, s1, gamma, beta):
    return pl.pallas_call(
        _final_body,
        out_shape=jax.ShapeDtypeStruct((N, D), jnp.float32),
    )(z, agg0, agg1, s0, s1, gamma.reshape(1, D), beta.reshape(1, D))


def kernel(z, edge_index, edge_type, type_emb, W1, b1, W2, b2,
           Wm1, bm1, Wm2, bm2, gamma, beta):
    src = edge_index[0]
    dst = edge_index[1]
    src3 = src.reshape(NW, NCHUNK, CH)
    dst3 = dst.reshape(NW, NCHUNK, CH)
    etf = edge_type.astype(jnp.float32).reshape(E, 1)

    ZMP = _node_mlp(z, Wm1, bm1, Wm2, bm2)
    ZS, ZDP = _sc_gather(z, ZMP, src3, dst3)
    ew2d, MSG = _edge_mlp(ZS, ZDP, etf, W1, b1, W2, b2, type_emb)
    ew3 = ew2d.reshape(NW, NCHUNK, CH)

    AGGP, SUMP = _sc_scatter(MSG, ew2d.reshape(E), src3)
    alpha3 = _sc_alpha(SUMP, ew3, src3)
    alpha = alpha3.reshape(E)
    out = _final_ln(z, AGGP[0, :N], AGGP[1, :N],
                    SUMP[0, :N].reshape(N, 1), SUMP[1, :N].reshape(N, 1),
                    gamma, beta)
    return (out, alpha)


# speedup vs baseline: 10.5912x; 1.0277x over previous
import functools

import jax
import jax.numpy as jnp
from jax import lax
from jax.experimental import pallas as pl
from jax.experimental.pallas import tpu as pltpu
from jax.experimental.pallas import tpu_sc as plsc

N = 10000
E = 320000
D = 128
TE = 8
T = 4000
ET_TILES = E // T

NC = 2
NS = 16
NW = NC * NS
EPW = E // NW
CH = 80
NCHUNK = EPW // CH
NPAIR = NCHUNK // 2
NG = EPW // 16
NPAD = 10240
SL = NPAD // NS

_mesh = plsc.VectorSubcoreMesh(core_axis_name="c", subcore_axis_name="s",
                               num_cores=NC, num_subcores=NS)
_sc_params = pltpu.CompilerParams(needs_layout_passes=False)


def _wid():
    return lax.axis_index("s") * NC + lax.axis_index("c")


def _node_mlp_body(z_ref, wm1_ref, bm1_ref, wm2_ref, bm2_ref, zmp_ref):
    z = z_ref[...]
    h = jnp.maximum(jnp.dot(z, wm1_ref[...], preferred_element_type=jnp.float32)
                    + bm1_ref[...], 0.0)
    m = jnp.dot(h, wm2_ref[...], preferred_element_type=jnp.float32) + bm2_ref[...]
    zu = lax.bitcast_convert_type(z.astype(jnp.bfloat16), jnp.uint16)
    mu = lax.bitcast_convert_type(m.astype(jnp.bfloat16), jnp.uint16)
    packed = zu.astype(jnp.uint32) | (mu.astype(jnp.uint32) << 16)
    zmp_ref[...] = lax.bitcast_convert_type(packed, jnp.int32)


def _node_mlp(z, Wm1, bm1, Wm2, bm2):
    return pl.pallas_call(
        _node_mlp_body,
        out_shape=jax.ShapeDtypeStruct((N, D), jnp.int32),
    )(z, Wm1, bm1.reshape(1, D), Wm2, bm2.reshape(1, D))


def _gather_body(z_hbm, zmp_hbm, src_hbm, dst_hbm, zs_out, zdp_out,
                 sidx_v, didx_v,
                 bs0, bs1, bs2, bs3, bd0, bd1, bd2, bd3,
                 gs0, gs1, gs2, gs3, gd0, gd1, gd2, gd3,
                 ws0, ws1, ws2, ws3, wd0, wd1, wd2, wd3):
    wid = _wid()
    pltpu.sync_copy(src_hbm.at[wid], sidx_v)
    pltpu.sync_copy(dst_hbm.at[wid], didx_v)
    base = wid * EPW
    BS = [bs0, bs1, bs2, bs3]
    BD = [bd0, bd1, bd2, bd3]
    GS = [gs0, gs1, gs2, gs3]
    GD = [gd0, gd1, gd2, gd3]
    WS = [ws0, ws1, ws2, ws3]
    WD = [wd0, wd1, wd2, wd3]

    def issue_g(i, j):
        pltpu.async_copy(z_hbm.at[sidx_v.at[j]], BS[i], GS[i])
        pltpu.async_copy(zmp_hbm.at[didx_v.at[j]], BD[i], GD[i])

    def wait_g(i, j):
        pltpu.make_async_copy(z_hbm.at[sidx_v.at[j]], BS[i], GS[i]).wait()
        pltpu.make_async_copy(zmp_hbm.at[didx_v.at[j]], BD[i], GD[i]).wait()

    def issue_w(i, j):
        row = pl.ds(base + j * CH, CH)
        pltpu.async_copy(BS[i], zs_out.at[row], WS[i])
        pltpu.async_copy(BD[i], zdp_out.at[row], WD[i])

    def wait_w(i):
        pltpu.make_async_copy(z_hbm.at[pl.ds(0, CH)], BS[i], WS[i]).wait()
        pltpu.make_async_copy(z_hbm.at[pl.ds(0, CH)], BD[i], WD[i]).wait()

    issue_g(0, 0)
    issue_g(1, 1)
    wait_g(0, 0)
    issue_w(0, 0)
    issue_g(2, 2)
    wait_g(1, 1)
    issue_w(1, 1)
    issue_g(3, 3)
    wait_g(2, 2)
    issue_w(2, 2)
    wait_w(0)
    issue_g(0, 4)
    wait_g(3, 3)
    issue_w(3, 3)
    wait_w(1)
    issue_g(1, 5)

    def body(k, carry):
        for i in range(4):
            j = 4 * k + i
            wait_g(i, j)
            issue_w(i, j)

            @pl.when(j + 2 < NCHUNK)
            def _():
                li = (i + 2) % 4
                wait_w(li)
                issue_g(li, j + 2)

        return carry

    lax.fori_loop(1, (NCHUNK - 1) // 4, body, 0)
    jlast = NCHUNK - 1
    wait_g(0, jlast)
    issue_w(0, jlast)
    wait_w(1)
    wait_w(2)
    wait_w(3)
    wait_w(0)


_sc_gather = functools.partial(
    pl.kernel,
    _gather_body,
    out_type=(jax.ShapeDtypeStruct((E, D), jnp.float32),
              jax.ShapeDtypeStruct((E, D), jnp.int32)),
    mesh=_mesh,
    compiler_params=_sc_params,
    scratch_types=[
        pltpu.VMEM((NCHUNK, CH), jnp.int32),
        pltpu.VMEM((NCHUNK, CH), jnp.int32),
        pltpu.VMEM((CH, D), jnp.float32),
        pltpu.VMEM((CH, D), jnp.float32),
        pltpu.VMEM((CH, D), jnp.float32),
        pltpu.VMEM((CH, D), jnp.float32),
        pltpu.VMEM((CH, D), jnp.int32),
        pltpu.VMEM((CH, D), jnp.int32),
        pltpu.VMEM((CH, D), jnp.int32),
        pltpu.VMEM((CH, D), jnp.int32),
    ] + [pltpu.SemaphoreType.DMA] * 16,
)()


def _edge_mlp_body(zs_ref, zdp_ref, etf_ref, w1_ref, b1_ref, w2_ref,
                   b2_ref, temb_ref, ew_ref, msg_ref):
    zs = zs_ref[...]
    word = lax.bitcast_convert_type(zdp_ref[...], jnp.uint32)
    zd = lax.bitcast_convert_type(word.astype(jnp.uint16), jnp.bfloat16)
    md = lax.bitcast_convert_type((word >> 16).astype(jnp.uint16), jnp.bfloat16)
    zdf = zd.astype(jnp.float32)
    d = jnp.abs(zs - zdf).astype(jnp.bfloat16)
    w1a = w1_ref[0:D, :].astype(jnp.bfloat16)
    w1b = w1_ref[D:2 * D, :].astype(jnp.bfloat16)
    w1c = w1_ref[2 * D:3 * D, :].astype(jnp.bfloat16)
    w1d = w1_ref[3 * D:3 * D + TE, :]
    t2 = jnp.dot(temb_ref[...], w1d, preferred_element_type=jnp.float32)
    pre = (jnp.dot(zs.astype(jnp.bfloat16), w1a, preferred_element_type=jnp.float32)
           + jnp.dot(zd, w1b, preferred_element_type=jnp.float32)
           + jnp.dot(d, w1c, preferred_element_type=jnp.float32)
           + b1_ref[...]
           + t2[0:1, :]
           + etf_ref[...] * (t2[1:2, :] - t2[0:1, :]))
    h = jnp.maximum(pre, 0.0)
    logit = jnp.dot(h, w2_ref[...], preferred_element_type=jnp.float32) + b2_ref[...]
    ew = jnp.exp(jax.nn.sigmoid(logit))
    ew_ref[...] = ew
    msg_ref[...] = ew * md.astype(jnp.float32)


def _edge_mlp(ZS, ZDP, etf, W1, b1, W2, b2, type_emb):
    return pl.pallas_call(
        _edge_mlp_body,
        grid=(ET_TILES,),
        in_specs=[
            pl.BlockSpec((T, D), lambda i: (i, 0)),
            pl.BlockSpec((T, D), lambda i: (i, 0)),
            pl.BlockSpec((T, 1), lambda i: (i, 0)),
            pl.BlockSpec((3 * D + TE, D), lambda i: (0, 0)),
            pl.BlockSpec((1, D), lambda i: (0, 0)),
            pl.BlockSpec((D, 1), lambda i: (0, 0)),
            pl.BlockSpec((1, 1), lambda i: (0, 0)),
            pl.BlockSpec((2, TE), lambda i: (0, 0)),
        ],
        out_specs=[
            pl.BlockSpec((T, 1), lambda i: (i, 0)),
            pl.BlockSpec((T, D), lambda i: (i, 0)),
        ],
        out_shape=[
            jax.ShapeDtypeStruct((E, 1), jnp.float32),
            jax.ShapeDtypeStruct((E, D), jnp.float32),
        ],
    )(ZS, ZDP, etf, W1, b1.reshape(1, D), W2, b2.reshape(1, 1), type_emb)


def _alpha_body(sump_hbm, ew_hbm, src_hbm, alpha_out, s2_v, sum_v, src_v,
                ew_v, al_v):
    wid = _wid()
    pltpu.sync_copy(sump_hbm, s2_v)

    def comb(i, carry):
        sl16 = pl.ds(i * 16, 16)
        sum_v[sl16] = s2_v[0, sl16] + s2_v[1, sl16]
        return carry

    lax.fori_loop(0, NPAD // 16, comb, 0)

    pltpu.sync_copy(src_hbm.at[wid], src_v)
    pltpu.sync_copy(ew_hbm.at[wid], ew_v)

    def body(r, carry):
        for g in range(CH // 16):
            sl16 = pl.ds(g * 16, 16)
            sg = plsc.load_gather(sum_v, [src_v[r, sl16]])
            al_v[r, sl16] = ew_v[r, sl16] / (sg + 1e-12)
        return carry

    lax.fori_loop(0, NCHUNK, body, 0)
    pltpu.sync_copy(al_v, alpha_out.at[wid])


_sc_alpha = functools.partial(
    pl.kernel,
    _alpha_body,
    out_type=jax.ShapeDtypeStruct((NW, NCHUNK, CH), jnp.float32),
    mesh=_mesh,
    compiler_params=_sc_params,
    scratch_types=[
        pltpu.VMEM((NC, NPAD), jnp.float32),
        pltpu.VMEM((NPAD,), jnp.float32),
        pltpu.VMEM((NCHUNK, CH), jnp.int32),
        pltpu.VMEM((NCHUNK, CH), jnp.float32),
        pltpu.VMEM((NCHUNK, CH), jnp.float32),
    ],
)()


def _scatter_body(msg_hbm, ew_hbm, src_hbm, aggp_out, sump_out,
                  src_v, mbuf0, mbuf1, eb0, eb1, zb_v, shared, sumsh,
                  sm0, sm1, se0, se1):
    cid = lax.axis_index("c")
    sid = lax.axis_index("s")
    wid = _wid()

    def zrow(i, carry):
        for g in range(D // 16):
            zb_v[i, pl.ds(g * 16, 16)] = jnp.zeros((16,), jnp.float32)
        return carry

    lax.fori_loop(0, CH, zrow, 0)
    for q in range(SL // CH):
        pltpu.sync_copy(zb_v, shared.at[pl.ds(sid * SL + q * CH, CH)])
    for q in range(SL // D):
        pltpu.sync_copy(zb_v.at[0], sumsh.at[pl.ds(sid * SL + q * D, D)])
    plsc.subcore_barrier()

    pltpu.sync_copy(src_hbm.at[wid], src_v)
    base = wid * EPW
    pltpu.async_copy(msg_hbm.at[pl.ds(base, CH)], mbuf0, sm0)
    pltpu.async_copy(ew_hbm.at[pl.ds(base, CH)], eb0, se0)
    pltpu.async_copy(msg_hbm.at[pl.ds(base + CH, CH)], mbuf1, sm1)
    pltpu.async_copy(ew_hbm.at[pl.ds(base + CH, CH)], eb1, se1)

    def step(j, mbuf, eb, sm, se):
        pltpu.make_async_copy(msg_hbm.at[pl.ds(base, CH)], mbuf, sm).wait()
        pltpu.make_async_copy(ew_hbm.at[pl.ds(base, CH)], eb, se).wait()
        pltpu.sync_copy(mbuf, shared.at[src_v.at[j]], add=True)
        pltpu.sync_copy(eb, sumsh.at[src_v.at[j]], add=True)

        @pl.when(j + 2 < NCHUNK)
        def _():
            pltpu.async_copy(msg_hbm.at[pl.ds(base + (j + 2) * CH, CH)],
                             mbuf, sm)
            pltpu.async_copy(ew_hbm.at[pl.ds(base + (j + 2) * CH, CH)],
                             eb, se)

    def body(k, carry):
        j0 = 2 * k
        step(j0, mbuf0, eb0, sm0, se0)
        step(j0 + 1, mbuf1, eb1, sm1, se1)
        return carry

    lax.fori_loop(0, NPAIR, body, 0)
    step(NCHUNK - 1, mbuf0, eb0, sm0, se0)
    plsc.subcore_barrier()
    pltpu.sync_copy(shared.at[pl.ds(sid * SL, SL)],
                    aggp_out.at[cid, pl.ds(sid * SL, SL)])
    pltpu.sync_copy(sumsh.at[pl.ds(sid * SL, SL)],
                    sump_out.at[cid, pl.ds(sid * SL, SL)])


_sc_scatter = functools.partial(
    pl.kernel,
    _scatter_body,
    out_type=(jax.ShapeDtypeStruct((NC, NPAD, D), jnp.float32),
              jax.ShapeDtypeStruct((NC, NPAD), jnp.float32)),
    mesh=_mesh,
    compiler_params=_sc_params,
    scratch_types=[
        pltpu.VMEM((NCHUNK, CH), jnp.int32),
        pltpu.VMEM((CH, D), jnp.float32),
        pltpu.VMEM((CH, D), jnp.float32),
        pltpu.VMEM((CH,), jnp.float32),
        pltpu.VMEM((CH,), jnp.float32),
        pltpu.VMEM((CH, D), jnp.float32),
        pltpu.VMEM_SHARED((NPAD, D), jnp.float32),
        pltpu.VMEM_SHARED((NPAD,), jnp.float32),
        pltpu.SemaphoreType.DMA,
        pltpu.SemaphoreType.DMA,
        pltpu.SemaphoreType.DMA,
        pltpu.SemaphoreType.DMA,
    ],
)()


def _final_body(z_ref, u0_ref, u1_ref, s0_ref, s1_ref, g_ref, b_ref, out_ref):
    agg = (u0_ref[...] + u1_ref[...]) / (s0_ref[...] + s1_ref[...] + 1e-12)
    x = z_ref[...] + agg
    mu = jnp.mean(x, axis=-1, keepdims=True)
    xc = x - mu
    var = jnp.mean(xc * xc, axis=-1, keepdims=True)
    out_ref[...] = xc * jax.lax.rsqrt(var + 1e-5) * g_ref[...] + b_ref[...]


def _final_ln(z, agg0, agg1, s0, s1, gamma, beta):
    return pl.pallas_call(
        _final_body,
        out_shape=jax.ShapeDtypeStruct((N, D), jnp.float32),
    )(z, agg0, agg1, s0, s1, gamma.reshape(1, D), beta.reshape(1, D))


def kernel(z, edge_index, edge_type, type_emb, W1, b1, W2, b2,
           Wm1, bm1, Wm2, bm2, gamma, beta):
    src = edge_index[0]
    dst = edge_index[1]
    src3 = src.reshape(NW, NCHUNK, CH)
    dst3 = dst.reshape(NW, NCHUNK, CH)
    etf = edge_type.astype(jnp.float32).reshape(E, 1)

    ZMP = _node_mlp(z, Wm1, bm1, Wm2, bm2)
    ZS, ZDP = _sc_gather(z, ZMP, src3, dst3)
    ew2d, MSG = _edge_mlp(ZS, ZDP, etf, W1, b1, W2, b2, type_emb)
    ew3 = ew2d.reshape(NW, NCHUNK, CH)

    AGGP, SUMP = _sc_scatter(MSG, ew2d.reshape(E), src3)
    alpha3 = _sc_alpha(SUMP, ew3, src3)
    alpha = alpha3.reshape(E)
    out = _final_ln(z, AGGP[0, :N], AGGP[1, :N],
                    SUMP[0, :N].reshape(N, 1), SUMP[1, :N].reshape(N, 1),
                    gamma, beta)
    return (out, alpha)


# speedup vs baseline: 10.6236x; 1.0031x over previous
import functools

import jax
import jax.numpy as jnp
from jax import lax
from jax.experimental import pallas as pl
from jax.experimental.pallas import tpu as pltpu
from jax.experimental.pallas import tpu_sc as plsc

N = 10000
E = 320000
D = 128
TE = 8
T = 8000
ET_TILES = E // T

NC = 2
NS = 16
NW = NC * NS
EPW = E // NW
CH = 80
NCHUNK = EPW // CH
NPAIR = NCHUNK // 2
NG = EPW // 16
NPAD = 10240
SL = NPAD // NS

_mesh = plsc.VectorSubcoreMesh(core_axis_name="c", subcore_axis_name="s",
                               num_cores=NC, num_subcores=NS)
_sc_params = pltpu.CompilerParams(needs_layout_passes=False)


def _wid():
    return lax.axis_index("s") * NC + lax.axis_index("c")


def _node_mlp_body(z_ref, wm1_ref, bm1_ref, wm2_ref, bm2_ref, zmp_ref):
    z = z_ref[...]
    h = jnp.maximum(jnp.dot(z, wm1_ref[...], preferred_element_type=jnp.float32)
                    + bm1_ref[...], 0.0)
    m = jnp.dot(h, wm2_ref[...], preferred_element_type=jnp.float32) + bm2_ref[...]
    zu = lax.bitcast_convert_type(z.astype(jnp.bfloat16), jnp.uint16)
    mu = lax.bitcast_convert_type(m.astype(jnp.bfloat16), jnp.uint16)
    packed = zu.astype(jnp.uint32) | (mu.astype(jnp.uint32) << 16)
    zmp_ref[...] = lax.bitcast_convert_type(packed, jnp.int32)


def _node_mlp(z, Wm1, bm1, Wm2, bm2):
    return pl.pallas_call(
        _node_mlp_body,
        out_shape=jax.ShapeDtypeStruct((N, D), jnp.int32),
    )(z, Wm1, bm1.reshape(1, D), Wm2, bm2.reshape(1, D))


def _gather_body(z_hbm, zmp_hbm, src_hbm, dst_hbm, zs_out, zdp_out,
                 sidx_v, didx_v,
                 bs0, bs1, bs2, bs3, bd0, bd1, bd2, bd3,
                 gs0, gs1, gs2, gs3, gd0, gd1, gd2, gd3,
                 ws0, ws1, ws2, ws3, wd0, wd1, wd2, wd3):
    wid = _wid()
    pltpu.sync_copy(src_hbm.at[wid], sidx_v)
    pltpu.sync_copy(dst_hbm.at[wid], didx_v)
    base = wid * EPW
    BS = [bs0, bs1, bs2, bs3]
    BD = [bd0, bd1, bd2, bd3]
    GS = [gs0, gs1, gs2, gs3]
    GD = [gd0, gd1, gd2, gd3]
    WS = [ws0, ws1, ws2, ws3]
    WD = [wd0, wd1, wd2, wd3]

    def issue_g(i, j):
        pltpu.async_copy(z_hbm.at[sidx_v.at[j]], BS[i], GS[i])
        pltpu.async_copy(zmp_hbm.at[didx_v.at[j]], BD[i], GD[i])

    def wait_g(i, j):
        pltpu.make_async_copy(z_hbm.at[sidx_v.at[j]], BS[i], GS[i]).wait()
        pltpu.make_async_copy(zmp_hbm.at[didx_v.at[j]], BD[i], GD[i]).wait()

    def issue_w(i, j):
        row = pl.ds(base + j * CH, CH)
        pltpu.async_copy(BS[i], zs_out.at[row], WS[i])
        pltpu.async_copy(BD[i], zdp_out.at[row], WD[i])

    def wait_w(i):
        pltpu.make_async_copy(z_hbm.at[pl.ds(0, CH)], BS[i], WS[i]).wait()
        pltpu.make_async_copy(z_hbm.at[pl.ds(0, CH)], BD[i], WD[i]).wait()

    issue_g(0, 0)
    issue_g(1, 1)
    wait_g(0, 0)
    issue_w(0, 0)
    issue_g(2, 2)
    wait_g(1, 1)
    issue_w(1, 1)
    issue_g(3, 3)
    wait_g(2, 2)
    issue_w(2, 2)
    wait_w(0)
    issue_g(0, 4)
    wait_g(3, 3)
    issue_w(3, 3)
    wait_w(1)
    issue_g(1, 5)

    def body(k, carry):
        for i in range(4):
            j = 4 * k + i
            wait_g(i, j)
            issue_w(i, j)

            @pl.when(j + 2 < NCHUNK)
            def _():
                li = (i + 2) % 4
                wait_w(li)
                issue_g(li, j + 2)

        return carry

    lax.fori_loop(1, (NCHUNK - 1) // 4, body, 0)
    jlast = NCHUNK - 1
    wait_g(0, jlast)
    issue_w(0, jlast)
    wait_w(1)
    wait_w(2)
    wait_w(3)
    wait_w(0)


_sc_gather = functools.partial(
    pl.kernel,
    _gather_body,
    out_type=(jax.ShapeDtypeStruct((E, D), jnp.float32),
              jax.ShapeDtypeStruct((E, D), jnp.int32)),
    mesh=_mesh,
    compiler_params=_sc_params,
    scratch_types=[
        pltpu.VMEM((NCHUNK, CH), jnp.int32),
        pltpu.VMEM((NCHUNK, CH), jnp.int32),
        pltpu.VMEM((CH, D), jnp.float32),
        pltpu.VMEM((CH, D), jnp.float32),
        pltpu.VMEM((CH, D), jnp.float32),
        pltpu.VMEM((CH, D), jnp.float32),
        pltpu.VMEM((CH, D), jnp.int32),
        pltpu.VMEM((CH, D), jnp.int32),
        pltpu.VMEM((CH, D), jnp.int32),
        pltpu.VMEM((CH, D), jnp.int32),
    ] + [pltpu.SemaphoreType.DMA] * 16,
)()


def _edge_mlp_body(zs_ref, zdp_ref, etf_ref, w1_ref, b1_ref, w2_ref,
                   b2_ref, temb_ref, ew_ref, msg_ref):
    zs = zs_ref[...]
    word = lax.bitcast_convert_type(zdp_ref[...], jnp.uint32)
    zd = lax.bitcast_convert_type(word.astype(jnp.uint16), jnp.bfloat16)
    md = lax.bitcast_convert_type((word >> 16).astype(jnp.uint16), jnp.bfloat16)
    zdf = zd.astype(jnp.float32)
    d = jnp.abs(zs - zdf).astype(jnp.bfloat16)
    w1a = w1_ref[0:D, :].astype(jnp.bfloat16)
    w1b = w1_ref[D:2 * D, :].astype(jnp.bfloat16)
    w1c = w1_ref[2 * D:3 * D, :].astype(jnp.bfloat16)
    w1d = w1_ref[3 * D:3 * D + TE, :]
    t2 = jnp.dot(temb_ref[...], w1d, preferred_element_type=jnp.float32)
    pre = (jnp.dot(zs.astype(jnp.bfloat16), w1a, preferred_element_type=jnp.float32)
           + jnp.dot(zd, w1b, preferred_element_type=jnp.float32)
           + jnp.dot(d, w1c, preferred_element_type=jnp.float32)
           + b1_ref[...]
           + t2[0:1, :]
           + etf_ref[...] * (t2[1:2, :] - t2[0:1, :]))
    h = jnp.maximum(pre, 0.0)
    logit = jnp.dot(h, w2_ref[...], preferred_element_type=jnp.float32) + b2_ref[...]
    ew = jnp.exp(jax.nn.sigmoid(logit))
    ew_ref[...] = ew
    msg_ref[...] = ew * md.astype(jnp.float32)


def _edge_mlp(ZS, ZDP, etf, W1, b1, W2, b2, type_emb):
    return pl.pallas_call(
        _edge_mlp_body,
        grid=(ET_TILES,),
        in_specs=[
            pl.BlockSpec((T, D), lambda i: (i, 0)),
            pl.BlockSpec((T, D), lambda i: (i, 0)),
            pl.BlockSpec((T, 1), lambda i: (i, 0)),
            pl.BlockSpec((3 * D + TE, D), lambda i: (0, 0)),
            pl.BlockSpec((1, D), lambda i: (0, 0)),
            pl.BlockSpec((D, 1), lambda i: (0, 0)),
            pl.BlockSpec((1, 1), lambda i: (0, 0)),
            pl.BlockSpec((2, TE), lambda i: (0, 0)),
        ],
        out_specs=[
            pl.BlockSpec((T, 1), lambda i: (i, 0)),
            pl.BlockSpec((T, D), lambda i: (i, 0)),
        ],
        out_shape=[
            jax.ShapeDtypeStruct((E, 1), jnp.float32),
            jax.ShapeDtypeStruct((E, D), jnp.float32),
        ],
    )(ZS, ZDP, etf, W1, b1.reshape(1, D), W2, b2.reshape(1, 1), type_emb)


def _alpha_body(sump_hbm, ew_hbm, src_hbm, alpha_out, s2_v, sum_v, src_v,
                ew_v, al_v):
    wid = _wid()
    pltpu.sync_copy(sump_hbm, s2_v)

    def comb(i, carry):
        sl16 = pl.ds(i * 16, 16)
        sum_v[sl16] = s2_v[0, sl16] + s2_v[1, sl16]
        return carry

    lax.fori_loop(0, NPAD // 16, comb, 0)

    pltpu.sync_copy(src_hbm.at[wid], src_v)
    pltpu.sync_copy(ew_hbm.at[wid], ew_v)

    def body(r, carry):
        for g in range(CH // 16):
            sl16 = pl.ds(g * 16, 16)
            sg = plsc.load_gather(sum_v, [src_v[r, sl16]])
            al_v[r, sl16] = ew_v[r, sl16] / (sg + 1e-12)
        return carry

    lax.fori_loop(0, NCHUNK, body, 0)
    pltpu.sync_copy(al_v, alpha_out.at[wid])


_sc_alpha = functools.partial(
    pl.kernel,
    _alpha_body,
    out_type=jax.ShapeDtypeStruct((NW, NCHUNK, CH), jnp.float32),
    mesh=_mesh,
    compiler_params=_sc_params,
    scratch_types=[
        pltpu.VMEM((NC, NPAD), jnp.float32),
        pltpu.VMEM((NPAD,), jnp.float32),
        pltpu.VMEM((NCHUNK, CH), jnp.int32),
        pltpu.VMEM((NCHUNK, CH), jnp.float32),
        pltpu.VMEM((NCHUNK, CH), jnp.float32),
    ],
)()


def _scatter_body(msg_hbm, ew_hbm, src_hbm, aggp_out, sump_out,
                  src_v, mbuf0, mbuf1, eb0, eb1, zb_v, shared, sumsh,
                  sm0, sm1, se0, se1):
    cid = lax.axis_index("c")
    sid = lax.axis_index("s")
    wid = _wid()

    def zrow(i, carry):
        for g in range(D // 16):
            zb_v[i, pl.ds(g * 16, 16)] = jnp.zeros((16,), jnp.float32)
        return carry

    lax.fori_loop(0, CH, zrow, 0)
    for q in range(SL // CH):
        pltpu.sync_copy(zb_v, shared.at[pl.ds(sid * SL + q * CH, CH)])
    for q in range(SL // D):
        pltpu.sync_copy(zb_v.at[0], sumsh.at[pl.ds(sid * SL + q * D, D)])
    plsc.subcore_barrier()

    pltpu.sync_copy(src_hbm.at[wid], src_v)
    base = wid * EPW
    pltpu.async_copy(msg_hbm.at[pl.ds(base, CH)], mbuf0, sm0)
    pltpu.async_copy(ew_hbm.at[pl.ds(base, CH)], eb0, se0)
    pltpu.async_copy(msg_hbm.at[pl.ds(base + CH, CH)], mbuf1, sm1)
    pltpu.async_copy(ew_hbm.at[pl.ds(base + CH, CH)], eb1, se1)

    def step(j, mbuf, eb, sm, se):
        pltpu.make_async_copy(msg_hbm.at[pl.ds(base, CH)], mbuf, sm).wait()
        pltpu.make_async_copy(ew_hbm.at[pl.ds(base, CH)], eb, se).wait()
        pltpu.sync_copy(mbuf, shared.at[src_v.at[j]], add=True)
        pltpu.sync_copy(eb, sumsh.at[src_v.at[j]], add=True)

        @pl.when(j + 2 < NCHUNK)
        def _():
            pltpu.async_copy(msg_hbm.at[pl.ds(base + (j + 2) * CH, CH)],
                             mbuf, sm)
            pltpu.async_copy(ew_hbm.at[pl.ds(base + (j + 2) * CH, CH)],
                             eb, se)

    def body(k, carry):
        j0 = 2 * k
        step(j0, mbuf0, eb0, sm0, se0)
        step(j0 + 1, mbuf1, eb1, sm1, se1)
        return carry

    lax.fori_loop(0, NPAIR, body, 0)
    step(NCHUNK - 1, mbuf0, eb0, sm0, se0)
    plsc.subcore_barrier()
    pltpu.sync_copy(shared.at[pl.ds(sid * SL, SL)],
                    aggp_out.at[cid, pl.ds(sid * SL, SL)])
    pltpu.sync_copy(sumsh.at[pl.ds(sid * SL, SL)],
                    sump_out.at[cid, pl.ds(sid * SL, SL)])


_sc_scatter = functools.partial(
    pl.kernel,
    _scatter_body,
    out_type=(jax.ShapeDtypeStruct((NC, NPAD, D), jnp.float32),
              jax.ShapeDtypeStruct((NC, NPAD), jnp.float32)),
    mesh=_mesh,
    compiler_params=_sc_params,
    scratch_types=[
        pltpu.VMEM((NCHUNK, CH), jnp.int32),
        pltpu.VMEM((CH, D), jnp.float32),
        pltpu.VMEM((CH, D), jnp.float32),
        pltpu.VMEM((CH,), jnp.float32),
        pltpu.VMEM((CH,), jnp.float32),
        pltpu.VMEM((CH, D), jnp.float32),
        pltpu.VMEM_SHARED((NPAD, D), jnp.float32),
        pltpu.VMEM_SHARED((NPAD,), jnp.float32),
        pltpu.SemaphoreType.DMA,
        pltpu.SemaphoreType.DMA,
        pltpu.SemaphoreType.DMA,
        pltpu.SemaphoreType.DMA,
    ],
)()


def _final_body(z_ref, u0_ref, u1_ref, s0_ref, s1_ref, g_ref, b_ref, out_ref):
    agg = (u0_ref[...] + u1_ref[...]) / (s0_ref[...] + s1_ref[...] + 1e-12)
    x = z_ref[...] + agg
    mu = jnp.mean(x, axis=-1, keepdims=True)
    xc = x - mu
    var = jnp.mean(xc * xc, axis=-1, keepdims=True)
    out_ref[...] = xc * jax.lax.rsqrt(var + 1e-5) * g_ref[...] + b_ref[...]


def _final_ln(z, agg0, agg1, s0, s1, gamma, beta):
    return pl.pallas_call(
        _final_body,
        out_shape=jax.ShapeDtypeStruct((N, D), jnp.float32),
    )(z, agg0, agg1, s0, s1, gamma.reshape(1, D), beta.reshape(1, D))


def kernel(z, edge_index, edge_type, type_emb, W1, b1, W2, b2,
           Wm1, bm1, Wm2, bm2, gamma, beta):
    src = edge_index[0]
    dst = edge_index[1]
    src3 = src.reshape(NW, NCHUNK, CH)
    dst3 = dst.reshape(NW, NCHUNK, CH)
    etf = edge_type.astype(jnp.float32).reshape(E, 1)

    ZMP = _node_mlp(z, Wm1, bm1, Wm2, bm2)
    ZS, ZDP = _sc_gather(z, ZMP, src3, dst3)
    ew2d, MSG = _edge_mlp(ZS, ZDP, etf, W1, b1, W2, b2, type_emb)
    ew3 = ew2d.reshape(NW, NCHUNK, CH)

    AGGP, SUMP = _sc_scatter(MSG, ew2d.reshape(E), src3)
    alpha3 = _sc_alpha(SUMP, ew3, src3)
    alpha = alpha3.reshape(E)
    out = _final_ln(z, AGGP[0, :N], AGGP[1, :N],
                    SUMP[0, :N].reshape(N, 1), SUMP[1, :N].reshape(N, 1),
                    gamma, beta)
    return (out, alpha)
